# baseline (speedup 1.0000x reference)
import functools

import jax
import jax.numpy as jnp
from jax import lax
from jax.experimental import pallas as pl
from jax.experimental.pallas import tpu as pltpu
from jax.experimental.pallas import tpu_sc as plsc

B = 16384
D = 32
H = 128
NC = 2
NS = 16
NW = NC * NS
BPW = B // NW
CH = 128
NCHUNK = BPW // CH


def _sc_gather_body(idx_d_hbm, idx_m_hbm, idx_p_hbm, Ed_hbm, Em_hbm, Ep_hbm,
                    out_d_hbm, out_m_hbm, out_p_hbm,
                    idx_v, rows_d, rows_m, rows_p, sem):
    wid = lax.axis_index("s") * NC + lax.axis_index("c")
    base = wid * BPW
    pltpu.sync_copy(idx_d_hbm.at[wid], idx_v.at[0])
    pltpu.sync_copy(idx_m_hbm.at[wid], idx_v.at[1])
    pltpu.sync_copy(idx_p_hbm.at[wid], idx_v.at[2])
    copies = []
    for table, rows, t in ((Ed_hbm, rows_d, 0), (Em_hbm, rows_m, 1),
                           (Ep_hbm, rows_p, 2)):
        for j in range(NCHUNK):
            copies.append(pltpu.async_copy(
                table.at[idx_v.at[t, j]],
                rows.at[pl.ds(j * CH, CH)], sem))
    for c in copies:
        c.wait()
    pltpu.sync_copy(rows_d, out_d_hbm.at[pl.ds(base, BPW)])
    pltpu.sync_copy(rows_m, out_m_hbm.at[pl.ds(base, BPW)])
    pltpu.sync_copy(rows_p, out_p_hbm.at[pl.ds(base, BPW)])


@functools.cache
def _sc_gather():
    return pl.kernel(
        _sc_gather_body,
        out_type=[jax.ShapeDtypeStruct((B, D), jnp.bfloat16)] * 3,
        mesh=plsc.VectorSubcoreMesh(core_axis_name="c", subcore_axis_name="s",
                                    num_cores=NC, num_subcores=NS),
        scratch_types=[
            pltpu.VMEM((3, NCHUNK, CH), jnp.int32),
            pltpu.VMEM((BPW, D), jnp.bfloat16),
            pltpu.VMEM((BPW, D), jnp.bfloat16),
            pltpu.VMEM((BPW, D), jnp.bfloat16),
            pltpu.SemaphoreType.DMA,
        ],
        compiler_params=pltpu.CompilerParams(use_tc_tiling_on_sc=False),
    )


BLK = 2048


def _proj_body(enc_ref, rd_ref, rm_ref, rp_ref,
               wenc_ref, wd_ref, wm_ref, wp_ref,
               benc_ref, bd_ref, bm_ref, bp_ref,
               oenc_ref, od_ref, om_ref, op_ref):
    def proj(x, w, b):
        y = jnp.dot(x, w, preferred_element_type=jnp.float32) + b
        return jnp.maximum(y, 0.0)
    oenc_ref[...] = proj(enc_ref[...], wenc_ref[...], benc_ref[...])
    od_ref[...] = proj(rd_ref[...].astype(jnp.float32), wd_ref[...], bd_ref[...])
    om_ref[...] = proj(rm_ref[...].astype(jnp.float32), wm_ref[...], bm_ref[...])
    op_ref[...] = proj(rp_ref[...].astype(jnp.float32), wp_ref[...], bp_ref[...])


def _project(encounter, rows_d, rows_m, rows_p,
             wenc_t, wd_t, wm_t, wp_t, benc, bd, bm, bp):
    grid = (B // BLK,)
    row_spec = pl.BlockSpec((BLK, D), lambda i: (i, 0))
    full = lambda s: pl.BlockSpec(s, lambda i: (0, 0))
    return pl.pallas_call(
        _proj_body,
        grid=grid,
        in_specs=[
            pl.BlockSpec((BLK, 128), lambda i: (i, 0)),
            row_spec, row_spec, row_spec,
            full((128, H)), full((D, H)), full((D, H)), full((D, H)),
            full((1, H)), full((1, H)), full((1, H)), full((1, H)),
        ],
        out_specs=[pl.BlockSpec((BLK, H), lambda i: (i, 0))] * 4,
        out_shape=[jax.ShapeDtypeStruct((B, H), jnp.float32)] * 4,
    )(encounter, rows_d, rows_m, rows_p,
      wenc_t, wd_t, wm_t, wp_t, benc, bd, bm, bp)


def kernel(encounter, diagnosis, medication, procedure,
           E_diag, E_med, E_proc,
           W_diag, b_diag, W_med, b_med, W_proc, b_proc,
           W_enc, b_enc):
    idx_d = diagnosis.astype(jnp.int32).reshape(NW, NCHUNK, CH)
    idx_m = medication.astype(jnp.int32).reshape(NW, NCHUNK, CH)
    idx_p = procedure.astype(jnp.int32).reshape(NW, NCHUNK, CH)
    rows_d, rows_m, rows_p = _sc_gather()(
        idx_d, idx_m, idx_p,
        E_diag.astype(jnp.bfloat16), E_med.astype(jnp.bfloat16),
        E_proc.astype(jnp.bfloat16))
    out_enc, out_d, out_m, out_p = _project(
        encounter, rows_d, rows_m, rows_p,
        W_enc.T, W_diag.T, W_med.T, W_proc.T,
        b_enc.reshape(1, H), b_diag.reshape(1, H),
        b_med.reshape(1, H), b_proc.reshape(1, H))
    return (out_enc, out_d, out_m, out_p)


# speedup vs baseline: 1.1198x; 1.1198x over previous
import functools

import jax
import jax.numpy as jnp
from jax import lax
from jax.experimental import pallas as pl
from jax.experimental.pallas import tpu as pltpu
from jax.experimental.pallas import tpu_sc as plsc

B = 16384
D = 32
H = 128
NC = 2
NS = 16
NW = NC * NS
BPW = B // NW
CH = 128
NCHUNK = BPW // CH
L = 16
NGRP = BPW // L


def _sc_gather_body(idx_d_hbm, idx_m_hbm, idx_p_hbm, Ed_hbm, Em_hbm, Ep_hbm,
                    out_d_hbm, out_m_hbm, out_p_hbm,
                    idx_v, idx4_v, rows4, rows, *sems):
    wid = lax.axis_index("s") * NC + lax.axis_index("c")
    base = wid * BPW
    pltpu.sync_copy(idx_d_hbm.at[wid], idx_v.at[0])
    pltpu.sync_copy(idx_m_hbm.at[wid], idx_v.at[1])
    pltpu.sync_copy(idx_p_hbm.at[wid], idx_v.at[2])

    iota = lax.iota(jnp.int32, L)

    for t, (table, out_hbm) in enumerate(
            ((Ed_hbm, out_d_hbm), (Em_hbm, out_m_hbm), (Ep_hbm, out_p_hbm))):
        for j in range(NCHUNK):
            for k in range(CH // L):
                sl = pl.ds(k * L, L)
                idx4_v[j, sl] = lax.shift_right_logical(idx_v[t, j, sl], 2)

        def fire(j, table=table):
            return pltpu.async_copy(table.at[idx4_v.at[j]],
                                    rows4.at[j % 2], sems[j % 2])

        pending = fire(0)
        for j in range(NCHUNK):
            pending.wait()
            if j + 1 < NCHUNK:
                pending = fire(j + 1)

            def extract(g, carry, t=t, j=j):
                v = idx_v[t, j, pl.ds(g * L, L)]
                off = lax.mul(lax.bitwise_and(v, jnp.int32(3)), jnp.int32(D))
                ridx = iota + g * L
                dst = ridx + j * CH
                for w in range(D):
                    x = plsc.load_gather(rows4.at[j % 2], [ridx, off + w])
                    plsc.store_scatter(
                        rows, [dst, jnp.full((L,), w, jnp.int32)], x)
                return carry

            lax.fori_loop(0, CH // L, extract, 0)
        pltpu.sync_copy(rows, out_hbm.at[pl.ds(base, BPW)])


@functools.cache
def _sc_gather():
    return pl.kernel(
        _sc_gather_body,
        out_type=[jax.ShapeDtypeStruct((B, D), jnp.float32)] * 3,
        mesh=plsc.VectorSubcoreMesh(core_axis_name="c", subcore_axis_name="s",
                                    num_cores=NC, num_subcores=NS),
        scratch_types=[
            pltpu.VMEM((3, NCHUNK, CH), jnp.int32),
            pltpu.VMEM((NCHUNK, CH), jnp.int32),
            pltpu.VMEM((2, CH, 4 * D), jnp.float32),
            pltpu.VMEM((BPW, D), jnp.float32),
            pltpu.SemaphoreType.DMA,
            pltpu.SemaphoreType.DMA,
        ],
        compiler_params=pltpu.CompilerParams(needs_layout_passes=False),
    )


BLK = 2048


def _proj_body(enc_ref, rd_ref, rm_ref, rp_ref,
               wenc_ref, wd_ref, wm_ref, wp_ref,
               benc_ref, bd_ref, bm_ref, bp_ref,
               oenc_ref, od_ref, om_ref, op_ref):
    def proj(x, w, b):
        y = jnp.dot(x, w, preferred_element_type=jnp.float32) + b
        return jnp.maximum(y, 0.0)
    oenc_ref[...] = proj(enc_ref[...], wenc_ref[...], benc_ref[...])
    od_ref[...] = proj(rd_ref[...], wd_ref[...], bd_ref[...])
    om_ref[...] = proj(rm_ref[...], wm_ref[...], bm_ref[...])
    op_ref[...] = proj(rp_ref[...], wp_ref[...], bp_ref[...])


def _project(encounter, rows_d, rows_m, rows_p,
             wenc_t, wd_t, wm_t, wp_t, benc, bd, bm, bp):
    grid = (B // BLK,)
    row_spec = pl.BlockSpec((BLK, D), lambda i: (i, 0))
    full = lambda s: pl.BlockSpec(s, lambda i: (0, 0))
    return pl.pallas_call(
        _proj_body,
        grid=grid,
        in_specs=[
            pl.BlockSpec((BLK, 128), lambda i: (i, 0)),
            row_spec, row_spec, row_spec,
            full((128, H)), full((D, H)), full((D, H)), full((D, H)),
            full((1, H)), full((1, H)), full((1, H)), full((1, H)),
        ],
        out_specs=[pl.BlockSpec((BLK, H), lambda i: (i, 0))] * 4,
        out_shape=[jax.ShapeDtypeStruct((B, H), jnp.float32)] * 4,
    )(encounter, rows_d, rows_m, rows_p,
      wenc_t, wd_t, wm_t, wp_t, benc, bd, bm, bp)


def kernel(encounter, diagnosis, medication, procedure,
           E_diag, E_med, E_proc,
           W_diag, b_diag, W_med, b_med, W_proc, b_proc,
           W_enc, b_enc):
    idx_d = diagnosis.astype(jnp.int32).reshape(NW, NCHUNK, CH)
    idx_m = medication.astype(jnp.int32).reshape(NW, NCHUNK, CH)
    idx_p = procedure.astype(jnp.int32).reshape(NW, NCHUNK, CH)
    rows_d, rows_m, rows_p = _sc_gather()(
        idx_d, idx_m, idx_p,
        E_diag.reshape(-1, 4 * D), E_med.reshape(-1, 4 * D),
        E_proc.reshape(-1, 4 * D))
    out_enc, out_d, out_m, out_p = _project(
        encounter, rows_d, rows_m, rows_p,
        W_enc.T, W_diag.T, W_med.T, W_proc.T,
        b_enc.reshape(1, H), b_diag.reshape(1, H),
        b_med.reshape(1, H), b_proc.reshape(1, H))
    return (out_enc, out_d, out_m, out_p)


# speedup vs baseline: 2.9592x; 2.6425x over previous
import functools

import jax
import jax.numpy as jnp
from jax import lax
from jax.experimental import pallas as pl
from jax.experimental.pallas import tpu as pltpu
from jax.experimental.pallas import tpu_sc as plsc

B = 16384
D = 32
H = 128
NC = 2
NS = 16
NW = NC * NS
BPW = B // NW
CH = 128
NCHUNK = BPW // CH
L = 16
NSLAB = 8


def _sc_diag_body(idx_hbm, table_hbm, out_hbm, idx_v, ring, rows, *sems):
    wid = lax.axis_index("s") * NC + lax.axis_index("c")
    base = wid * BPW
    pltpu.sync_copy(idx_hbm.at[wid], idx_v)
    iota = lax.iota(jnp.int32, L)

    def scalar_idx(b):
        chunk = idx_v[0, pl.ds((b // L) * L, L)]
        sel = jnp.where(iota == b % L, chunk, 0)
        return lax.reduce_max(sel, (0,))

    def fire(b, slot):
        v = scalar_idx(b)
        col = pl.multiple_of(
            lax.shift_left(lax.shift_right_logical(v, 7), 7), CH)
        pltpu.async_copy(table_hbm.at[:, pl.ds(col, CH)], ring.at[slot],
                         sems[slot])

    def extract(b, slot):
        pltpu.make_async_copy(table_hbm.at[:, pl.ds(0, CH)], ring.at[slot],
                              sems[slot]).wait()
        v = scalar_idx(b)
        lane = jnp.full((L,), lax.bitwise_and(v, jnp.int32(CH - 1)),
                        jnp.int32)
        for t in range(D // L):
            x = plsc.load_gather(ring.at[slot], [iota + t * L, lane])
            rows[b, pl.ds(t * L, L)] = x

    for r in range(NSLAB):
        fire(r, r)

    def step(o, carry, do_fire=True):
        for r in range(NSLAB):
            b = o * NSLAB + r
            extract(b, r)
            if do_fire:
                fire(b + NSLAB, r)
        return carry

    lax.fori_loop(0, BPW // NSLAB - 1, step, 0)
    step(BPW // NSLAB - 1, 0, do_fire=False)
    pltpu.sync_copy(rows, out_hbm.at[pl.ds(base, BPW)])


@functools.cache
def _sc_diag():
    return pl.kernel(
        _sc_diag_body,
        out_type=jax.ShapeDtypeStruct((B, H), jnp.float32),
        mesh=plsc.VectorSubcoreMesh(core_axis_name="c", subcore_axis_name="s",
                                    num_cores=NC, num_subcores=NS),
        scratch_types=[
            pltpu.VMEM((1, BPW), jnp.int32),
            pltpu.VMEM((NSLAB, D, CH), jnp.float32),
            pltpu.VMEM((BPW, H), jnp.float32),
        ] + [pltpu.SemaphoreType.DMA] * NSLAB,
        compiler_params=pltpu.CompilerParams(use_tc_tiling_on_sc=True,
                                             needs_layout_passes=False),
    )


def _sc_mp_body(idx_m_hbm, idx_p_hbm, Em_hbm, Ep_hbm,
                out_m_hbm, out_p_hbm, idx_v, rows_m, rows_p, sem):
    wid = lax.axis_index("s") * NC + lax.axis_index("c")
    base = wid * BPW
    pltpu.sync_copy(idx_m_hbm.at[wid], idx_v.at[0])
    pltpu.sync_copy(idx_p_hbm.at[wid], idx_v.at[1])
    copies = []
    for table, rows, t in ((Em_hbm, rows_m, 0), (Ep_hbm, rows_p, 1)):
        for j in range(NCHUNK):
            copies.append(pltpu.async_copy(
                table.at[idx_v.at[t, j]],
                rows.at[pl.ds(j * CH, CH)], sem))
    for c in copies:
        c.wait()
    pltpu.sync_copy(rows_m, out_m_hbm.at[pl.ds(base, BPW)])
    pltpu.sync_copy(rows_p, out_p_hbm.at[pl.ds(base, BPW)])


@functools.cache
def _sc_mp():
    return pl.kernel(
        _sc_mp_body,
        out_type=[jax.ShapeDtypeStruct((B, D), jnp.float32)] * 2,
        mesh=plsc.VectorSubcoreMesh(core_axis_name="c", subcore_axis_name="s",
                                    num_cores=NC, num_subcores=NS),
        scratch_types=[
            pltpu.VMEM((2, NCHUNK, CH), jnp.int32),
            pltpu.VMEM((BPW, D), jnp.float32),
            pltpu.VMEM((BPW, D), jnp.float32),
            pltpu.SemaphoreType.DMA,
        ],
        compiler_params=pltpu.CompilerParams(use_tc_tiling_on_sc=False),
    )


BLK = 2048


def _proj_body(enc_ref, rd_ref, rm_ref, rp_ref,
               wenc_ref, wd_ref, wm_ref, wp_ref,
               benc_ref, bd_ref, bm_ref, bp_ref,
               oenc_ref, od_ref, om_ref, op_ref):
    def proj(x, w, b):
        y = jnp.dot(x, w, preferred_element_type=jnp.float32) + b
        return jnp.maximum(y, 0.0)
    oenc_ref[...] = proj(enc_ref[...], wenc_ref[...], benc_ref[...])
    od_ref[...] = proj(rd_ref[:, :D], wd_ref[...], bd_ref[...])
    om_ref[...] = proj(rm_ref[...], wm_ref[...], bm_ref[...])
    op_ref[...] = proj(rp_ref[...], wp_ref[...], bp_ref[...])


def _project(encounter, rows_d, rows_m, rows_p,
             wenc_t, wd_t, wm_t, wp_t, benc, bd, bm, bp):
    grid = (B // BLK,)
    row_spec = pl.BlockSpec((BLK, D), lambda i: (i, 0))
    full = lambda s: pl.BlockSpec(s, lambda i: (0, 0))
    return pl.pallas_call(
        _proj_body,
        grid=grid,
        in_specs=[
            pl.BlockSpec((BLK, 128), lambda i: (i, 0)),
            pl.BlockSpec((BLK, H), lambda i: (i, 0)),
            row_spec, row_spec,
            full((128, H)), full((D, H)), full((D, H)), full((D, H)),
            full((1, H)), full((1, H)), full((1, H)), full((1, H)),
        ],
        out_specs=[pl.BlockSpec((BLK, H), lambda i: (i, 0))] * 4,
        out_shape=[jax.ShapeDtypeStruct((B, H), jnp.float32)] * 4,
    )(encounter, rows_d, rows_m, rows_p,
      wenc_t, wd_t, wm_t, wp_t, benc, bd, bm, bp)


def kernel(encounter, diagnosis, medication, procedure,
           E_diag, E_med, E_proc,
           W_diag, b_diag, W_med, b_med, W_proc, b_proc,
           W_enc, b_enc):
    idx_d = diagnosis.astype(jnp.int32).reshape(NW, 1, BPW)
    idx_m = medication.astype(jnp.int32).reshape(NW, NCHUNK, CH)
    idx_p = procedure.astype(jnp.int32).reshape(NW, NCHUNK, CH)
    rows_d = _sc_diag()(idx_d, E_diag.T)
    rows_m, rows_p = _sc_mp()(idx_m, idx_p, E_med, E_proc)
    out_enc, out_d, out_m, out_p = _project(
        encounter, rows_d, rows_m, rows_p,
        W_enc.T, W_diag.T, W_med.T, W_proc.T,
        b_enc.reshape(1, H), b_diag.reshape(1, H),
        b_med.reshape(1, H), b_proc.reshape(1, H))
    return (out_enc, out_d, out_m, out_p)


# speedup vs baseline: 2.9630x; 1.0013x over previous
import functools

import jax
import jax.numpy as jnp
from jax import lax
from jax.experimental import pallas as pl
from jax.experimental.pallas import tpu as pltpu
from jax.experimental.pallas import tpu_sc as plsc

B = 16384
D = 32
H = 128
NC = 2
NS = 16
NW = NC * NS
BPW = B // NW
CH = 128
NCHUNK = BPW // CH
L = 16
NSLAB = 8


def _sc_diag_body(idx_hbm, table_hbm, out_hbm, idx_v, ring, rows, *sems):
    wid = lax.axis_index("s") * NC + lax.axis_index("c")
    base = wid * BPW
    pltpu.sync_copy(idx_hbm.at[wid], idx_v)
    iota = lax.iota(jnp.int32, L)

    def scalar_idx(b):
        chunk = idx_v[0, pl.ds((b // L) * L, L)]
        sel = jnp.where(iota == b % L, chunk, 0)
        return lax.reduce_max(sel, (0,))

    def fire(b, slot):
        v = scalar_idx(b)
        col = pl.multiple_of(
            lax.shift_left(lax.shift_right_logical(v, 7), 7), CH)
        pltpu.async_copy(table_hbm.at[:, pl.ds(col, CH)], ring.at[slot],
                         sems[slot])

    def extract(b, slot):
        pltpu.make_async_copy(table_hbm.at[:, pl.ds(0, CH)], ring.at[slot],
                              sems[slot]).wait()
        v = scalar_idx(b)
        lane = jnp.full((L,), lax.bitwise_and(v, jnp.int32(CH - 1)),
                        jnp.int32)
        for t in range(D // L):
            x = plsc.load_gather(ring.at[slot], [iota + t * L, lane])
            rows[b, pl.ds(t * L, L)] = x

    for r in range(NSLAB):
        fire(r, r)

    def step(o, carry, do_fire=True):
        for r in range(NSLAB):
            b = o * NSLAB + r
            extract(b, r)
            if do_fire:
                fire(b + NSLAB, r)
        return carry

    lax.fori_loop(0, BPW // NSLAB - 1, step, 0)
    step(BPW // NSLAB - 1, 0, do_fire=False)
    pltpu.sync_copy(rows, out_hbm.at[pl.ds(base, BPW)])


@functools.cache
def _sc_diag():
    return pl.kernel(
        _sc_diag_body,
        out_type=jax.ShapeDtypeStruct((B, H), jnp.float32),
        mesh=plsc.VectorSubcoreMesh(core_axis_name="c", subcore_axis_name="s",
                                    num_cores=NC, num_subcores=NS),
        scratch_types=[
            pltpu.VMEM((1, BPW), jnp.int32),
            pltpu.VMEM((NSLAB, D, CH), jnp.float32),
            pltpu.VMEM((BPW, H), jnp.float32),
        ] + [pltpu.SemaphoreType.DMA] * NSLAB,
        compiler_params=pltpu.CompilerParams(use_tc_tiling_on_sc=True,
                                             needs_layout_passes=False),
    )


def _sc_mp_body(idx_m_hbm, idx_p_hbm, Em_hbm, Ep_hbm,
                out_m_hbm, out_p_hbm, idx_v, rows_m, rows_p, sem):
    wid = lax.axis_index("s") * NC + lax.axis_index("c")
    base = wid * BPW
    pltpu.sync_copy(idx_m_hbm.at[wid], idx_v.at[0])
    pltpu.sync_copy(idx_p_hbm.at[wid], idx_v.at[1])
    copies = []
    for table, rows, t in ((Em_hbm, rows_m, 0), (Ep_hbm, rows_p, 1)):
        for j in range(NCHUNK):
            copies.append(pltpu.async_copy(
                table.at[idx_v.at[t, j]],
                rows.at[pl.ds(j * CH, CH)], sem))
    for c in copies:
        c.wait()
    pltpu.sync_copy(rows_m, out_m_hbm.at[pl.ds(base, BPW)])
    pltpu.sync_copy(rows_p, out_p_hbm.at[pl.ds(base, BPW)])


@functools.cache
def _sc_mp():
    return pl.kernel(
        _sc_mp_body,
        out_type=[jax.ShapeDtypeStruct((B, D), jnp.float32)] * 2,
        mesh=plsc.VectorSubcoreMesh(core_axis_name="c", subcore_axis_name="s",
                                    num_cores=NC, num_subcores=NS),
        scratch_types=[
            pltpu.VMEM((2, NCHUNK, CH), jnp.int32),
            pltpu.VMEM((BPW, D), jnp.float32),
            pltpu.VMEM((BPW, D), jnp.float32),
            pltpu.SemaphoreType.DMA,
        ],
        compiler_params=pltpu.CompilerParams(use_tc_tiling_on_sc=False),
    )


BLK = 2048


def _proj_body(enc_ref, rd_ref, rm_ref, rp_ref,
               wenc_ref, wd_ref, wm_ref, wp_ref,
               benc_ref, bd_ref, bm_ref, bp_ref,
               oenc_ref, od_ref, om_ref, op_ref):
    def proj(x, w, b):
        y = jnp.dot(x, w, preferred_element_type=jnp.float32) + b
        return jnp.maximum(y, 0.0)
    oenc_ref[...] = proj(enc_ref[...], wenc_ref[...], benc_ref[...])
    od_ref[...] = proj(rd_ref[:, :D], wd_ref[...], bd_ref[...])
    om_ref[...] = proj(rm_ref[...], wm_ref[...], bm_ref[...])
    op_ref[...] = proj(rp_ref[...], wp_ref[...], bp_ref[...])


def _project(encounter, rows_d, rows_m, rows_p,
             wenc_t, wd_t, wm_t, wp_t, benc, bd, bm, bp):
    grid = (B // BLK,)
    row_spec = pl.BlockSpec((BLK, D), lambda i: (i, 0))
    full = lambda s: pl.BlockSpec(s, lambda i: (0, 0))
    return pl.pallas_call(
        _proj_body,
        grid=grid,
        in_specs=[
            pl.BlockSpec((BLK, 128), lambda i: (i, 0)),
            pl.BlockSpec((BLK, H), lambda i: (i, 0)),
            row_spec, row_spec,
            full((128, H)), full((D, H)), full((D, H)), full((D, H)),
            full((1, H)), full((1, H)), full((1, H)), full((1, H)),
        ],
        out_specs=[pl.BlockSpec((BLK, H), lambda i: (i, 0))] * 4,
        out_shape=[jax.ShapeDtypeStruct((B, H), jnp.float32)] * 4,
    )(encounter, rows_d, rows_m, rows_p,
      wenc_t, wd_t, wm_t, wp_t, benc, bd, bm, bp)


def kernel(encounter, diagnosis, medication, procedure,
           E_diag, E_med, E_proc,
           W_diag, b_diag, W_med, b_med, W_proc, b_proc,
           W_enc, b_enc):
    idx_d = diagnosis.astype(jnp.int32).reshape(NW, 1, BPW)
    idx_m = medication.astype(jnp.int32).reshape(NW, NCHUNK, CH)
    idx_p = procedure.astype(jnp.int32).reshape(NW, NCHUNK, CH)
    rows_m, rows_p = _sc_mp()(idx_m, idx_p, E_med, E_proc)
    rows_d = _sc_diag()(idx_d, E_diag.T)
    out_enc, out_d, out_m, out_p = _project(
        encounter, rows_d, rows_m, rows_p,
        W_enc.T, W_diag.T, W_med.T, W_proc.T,
        b_enc.reshape(1, H), b_diag.reshape(1, H),
        b_med.reshape(1, H), b_proc.reshape(1, H))
    return (out_enc, out_d, out_m, out_p)
